# baseline (speedup 1.0000x reference)
import functools

import jax
import jax.numpy as jnp
from jax import lax
from jax.experimental import pallas as pl
from jax.experimental.pallas import tpu as pltpu
from jax.experimental.pallas import tpu_sc as plsc

T = 4096
D = 1024
F = 512
E = 8
K = 2

BT = 512
BG = 512
G = T * K // BG + E
P = G * BG

NC = 2
NS = 16
NW = NC * NS
CH = 32


def _router_body(hs_ref, rw_ref, w8_ref, dst0_ref, dst1_ref, te_ref,
                 cnt_ref, e0_ref, e1_ref, r0_ref, r1_ref):
    i = pl.program_id(0)
    nsteps = pl.num_programs(0)

    @pl.when(i == 0)
    def _init():
        cnt_ref[...] = jnp.zeros((1, E), jnp.float32)

    x = hs_ref[...]
    logits = lax.dot_general(x, rw_ref[...], (((1,), (1,)), ((), ())),
                             preferred_element_type=jnp.float32)
    m = jnp.max(logits, axis=-1, keepdims=True)
    ex = jnp.exp(logits - m)
    probs = ex / jnp.sum(ex, axis=-1, keepdims=True)
    cols = lax.broadcasted_iota(jnp.int32, (BT, E), 1)
    i1 = jnp.argmax(probs, axis=-1, keepdims=True)
    is1 = cols == i1
    probs_m = jnp.where(is1, -1.0, probs)
    i2 = jnp.argmax(probs_m, axis=-1, keepdims=True)
    is2 = cols == i2
    w1 = jnp.max(probs, axis=-1, keepdims=True)
    w2 = jnp.max(probs_m, axis=-1, keepdims=True)

    w8_ref[...] = jnp.where(cols == 0, w1, jnp.where(cols == 1, w2, 0.0))

    a1 = is1.astype(jnp.float32)
    a2 = is2.astype(jnp.float32)
    tri = (lax.broadcasted_iota(jnp.int32, (BT, BT), 1)
           < lax.broadcasted_iota(jnp.int32, (BT, BT), 0)).astype(jnp.bfloat16)
    cum1 = jnp.dot(tri, a1.astype(jnp.bfloat16),
                   preferred_element_type=jnp.float32)
    cum2 = jnp.dot(tri, a2.astype(jnp.bfloat16),
                   preferred_element_type=jnp.float32)
    s1 = jnp.sum(a1, axis=0, keepdims=True)
    s2 = jnp.sum(a2, axis=0, keepdims=True)
    cnt = cnt_ref[...]
    rank1 = cnt + cum1
    rank2 = cnt + s1 + cum2
    cnt_ref[...] = cnt + s1 + s2

    e0_ref[i] = a1
    e1_ref[i] = a2
    r0_ref[i] = a1 * rank1
    r1_ref[i] = a2 * rank2

    @pl.when(i == nsteps - 1)
    def _finale():
        counts = cnt_ref[...]
        tiles = jnp.ceil(counts * (1.0 / BG))
        et = lax.broadcasted_iota(jnp.int32, (E, E), 0)
        ee = lax.broadcasted_iota(jnp.int32, (E, E), 1)
        mlt = (et < ee).astype(jnp.float32)
        cum_excl = jnp.dot(tiles, mlt,
                           preferred_element_type=jnp.float32)
        base_rows = cum_excl * float(BG)

        oh0 = e0_ref[...].reshape(T, E)
        oh1 = e1_ref[...].reshape(T, E)
        rf0 = r0_ref[...].reshape(T, E)
        rf1 = r1_ref[...].reshape(T, E)
        dst0_ref[...] = jnp.sum(oh0 * base_rows + rf0, axis=-1,
                                keepdims=True).astype(jnp.int32)
        dst1_ref[...] = jnp.sum(oh1 * base_rows + rf1, axis=-1,
                                keepdims=True).astype(jnp.int32)

        gg = lax.broadcasted_iota(jnp.int32, (G, E), 0).astype(jnp.float32)
        ge = lax.broadcasted_iota(jnp.int32, (G, E), 1).astype(jnp.float32)
        lo = cum_excl
        hi = cum_excl + tiles
        in_e = jnp.logical_and(gg >= lo, gg < hi).astype(jnp.float32)
        te = jnp.sum(in_e * ge, axis=-1, keepdims=True)
        any_e = jnp.sum(in_e, axis=-1, keepdims=True)
        te_ref[...] = jnp.where(any_e > 0.0, te, -1.0).astype(jnp.int32)


def _router(hidden_states, router_weight):
    return pl.pallas_call(
        _router_body,
        grid=(T // BT,),
        in_specs=[
            pl.BlockSpec((BT, D), lambda i: (i, 0)),
            pl.BlockSpec((E, D), lambda i: (0, 0)),
        ],
        out_specs=[
            pl.BlockSpec((BT, E), lambda i: (i, 0)),
            pl.BlockSpec((T, 1), lambda i: (0, 0)),
            pl.BlockSpec((T, 1), lambda i: (0, 0)),
            pl.BlockSpec((G, 1), lambda i: (0, 0)),
        ],
        out_shape=[
            jax.ShapeDtypeStruct((T, E), jnp.float32),
            jax.ShapeDtypeStruct((T, 1), jnp.int32),
            jax.ShapeDtypeStruct((T, 1), jnp.int32),
            jax.ShapeDtypeStruct((G, 1), jnp.int32),
        ],
        scratch_shapes=[
            pltpu.VMEM((1, E), jnp.float32),
            pltpu.VMEM((T // BT, BT, E), jnp.float32),
            pltpu.VMEM((T // BT, BT, E), jnp.float32),
            pltpu.VMEM((T // BT, BT, E), jnp.float32),
            pltpu.VMEM((T // BT, BT, E), jnp.float32),
        ],
        compiler_params=pltpu.CompilerParams(
            dimension_semantics=("arbitrary",)),
    )(hidden_states, router_weight)


@functools.cache
def _make_dispatch():
    mesh = plsc.VectorSubcoreMesh(core_axis_name="c", subcore_axis_name="s")

    nch = T * K // NW // CH

    @functools.partial(
        pl.kernel,
        out_type=jax.ShapeDtypeStruct((P, D), jnp.float32),
        mesh=mesh,
        scratch_types=[
            pltpu.VMEM((nch, CH), jnp.int32),
            pltpu.VMEM((CH, D), jnp.float32),
            pltpu.VMEM((CH, D), jnp.float32),
            pltpu.SemaphoreType.DMA,
            pltpu.SemaphoreType.DMA,
            pltpu.SemaphoreType.DMA,
        ],
    )
    def _dispatch(hid_hbm, idx_hbm, xp_hbm, idx_v, rows0, rows1, ls0, ls1,
                  ss):
        wid = lax.axis_index("s") * NC + lax.axis_index("c")
        npairs = T * K // NW
        tb = (wid % (T // npairs)) * npairs
        rows = (rows0, rows1)
        lsem = (ls0, ls1)
        pltpu.sync_copy(idx_hbm.at[pl.ds(wid * nch, nch)], idx_v)
        cp0 = pltpu.make_async_copy(hid_hbm.at[pl.ds(tb, CH)], rows0, ls0)
        cp0.start()
        cp1 = pltpu.make_async_copy(hid_hbm.at[pl.ds(tb + CH, CH)], rows1,
                                    ls1)
        cp1.start()
        loads = [cp0, cp1]
        for c in range(nch):
            b = c % 2
            loads[b].wait()
            pltpu.async_copy(rows[b], xp_hbm.at[idx_v.at[c]], ss).wait()
            if c + 2 < nch:
                cp = pltpu.make_async_copy(
                    hid_hbm.at[pl.ds(tb + (c + 2) * CH, CH)], rows[b],
                    lsem[b])
                cp.start()
                loads[b] = cp

    return _dispatch


def _gemm_body(te_ref, x_ref, wgu_ref, wd_ref, y_ref):
    g = pl.program_id(0)

    @pl.when(te_ref[g] >= 0)
    def _compute():
        gu = jnp.dot(x_ref[...].astype(jnp.bfloat16), wgu_ref[0],
                     preferred_element_type=jnp.float32)
        gt = gu[:, :F]
        up = gu[:, F:]
        h = (gt * lax.logistic(gt) * up).astype(jnp.bfloat16)
        y_ref[...] = jnp.dot(h, wd_ref[0],
                             preferred_element_type=jnp.float32)


def _cast_body(wgu_ref, wd_ref, wgu_bf_ref, wd_bf_ref):
    wgu_bf_ref[...] = wgu_ref[...].astype(jnp.bfloat16)
    wd_bf_ref[...] = wd_ref[...].astype(jnp.bfloat16)


def _cast_weights(wgu, wd):
    return pl.pallas_call(
        _cast_body,
        grid=(E,),
        in_specs=[
            pl.BlockSpec((1, D, 2 * F), lambda e: (e, 0, 0)),
            pl.BlockSpec((1, F, D), lambda e: (e, 0, 0)),
        ],
        out_specs=[
            pl.BlockSpec((1, D, 2 * F), lambda e: (e, 0, 0)),
            pl.BlockSpec((1, F, D), lambda e: (e, 0, 0)),
        ],
        out_shape=[
            jax.ShapeDtypeStruct((E, D, 2 * F), jnp.bfloat16),
            jax.ShapeDtypeStruct((E, F, D), jnp.bfloat16),
        ],
    )(wgu, wd)


def _gemm(te, xp, wgu_bf, wd_bf):
    def _emap(g, te):
        return (jnp.where(te[g] < 0, E - 1, te[g]), 0, 0)

    grid_spec = pltpu.PrefetchScalarGridSpec(
        num_scalar_prefetch=1,
        grid=(G,),
        in_specs=[
            pl.BlockSpec((BG, D), lambda g, te: (g, 0)),
            pl.BlockSpec((1, D, 2 * F), _emap),
            pl.BlockSpec((1, F, D), _emap),
        ],
        out_specs=pl.BlockSpec((BG, D), lambda g, te: (g, 0)),
    )
    return pl.pallas_call(
        _gemm_body,
        grid_spec=grid_spec,
        out_shape=jax.ShapeDtypeStruct((P, D), jnp.float32),
        compiler_params=pltpu.CompilerParams(
            dimension_semantics=("arbitrary",)),
    )(te, xp, wgu_bf, wd_bf)


@functools.cache
def _make_combine():
    mesh = plsc.VectorSubcoreMesh(core_axis_name="c", subcore_axis_name="s")

    nch = T // NW // CH

    @functools.partial(
        pl.kernel,
        out_type=[
            jax.ShapeDtypeStruct((T, D), jnp.float32),
            jax.ShapeDtypeStruct((T, D), jnp.float32),
        ],
        mesh=mesh,
        scratch_types=[
            pltpu.VMEM((nch, CH), jnp.int32),
            pltpu.VMEM((nch, CH), jnp.int32),
            pltpu.VMEM((CH, D), jnp.float32),
            pltpu.VMEM((CH, D), jnp.float32),
            pltpu.SemaphoreType.DMA,
            pltpu.SemaphoreType.DMA,
        ],
    )
    def _combine(y_hbm, d0_hbm, d1_hbm, a_hbm, b_hbm, i0_v, i1_v, rows0,
                 rows1, g0, g1, ):
        wid = lax.axis_index("s") * NC + lax.axis_index("c")
        ntok = T // NW
        rows = (rows0, rows1)
        gsem = (g0, g1)
        pltpu.sync_copy(d0_hbm.at[pl.ds(wid * nch, nch)], i0_v)
        pltpu.sync_copy(d1_hbm.at[pl.ds(wid * nch, nch)], i1_v)
        units = [(kk, cc) for cc in range(nch) for kk in range(2)]

        def _start(u, b):
            kk, cc = units[u]
            iv = i0_v if kk == 0 else i1_v
            cp = pltpu.make_async_copy(y_hbm.at[iv.at[cc]], rows[b],
                                       gsem[b])
            cp.start()
            return cp

        gat = [_start(0, 0), _start(1, 1)]
        for u in range(len(units)):
            b = u % 2
            kk, cc = units[u]
            gat[b].wait()
            dest = a_hbm if kk == 0 else b_hbm
            pltpu.sync_copy(rows[b], dest.at[pl.ds(wid * ntok + cc * CH,
                                                   CH)])
            if u + 2 < len(units):
                gat[b] = _start(u + 2, b)

    return _combine


def _final_body(a_ref, b_ref, w8_ref, out_ref):
    w8 = w8_ref[...]
    cols = lax.broadcasted_iota(jnp.int32, (BT, E), 1)
    w0 = jnp.sum(jnp.where(cols == 0, w8, 0.0), axis=-1, keepdims=True)
    w1 = jnp.sum(jnp.where(cols == 1, w8, 0.0), axis=-1, keepdims=True)
    out_ref[...] = w0 * a_ref[...] + w1 * b_ref[...]


def _final(a, b, w8):
    return pl.pallas_call(
        _final_body,
        grid=(T // BT,),
        in_specs=[
            pl.BlockSpec((BT, D), lambda i: (i, 0)),
            pl.BlockSpec((BT, D), lambda i: (i, 0)),
            pl.BlockSpec((BT, E), lambda i: (i, 0)),
        ],
        out_specs=pl.BlockSpec((BT, D), lambda i: (i, 0)),
        out_shape=jax.ShapeDtypeStruct((T, D), jnp.float32),
    )(a, b, w8)


@jax.jit
def kernel(hidden_states, router_weight, merged_gate_up_proj, merged_down_proj):
    w8, dst0, dst1, te = _router(hidden_states, router_weight)
    wgu_bf, wd_bf = _cast_weights(merged_gate_up_proj, merged_down_proj)
    idx2 = jnp.concatenate([dst0, dst1], axis=0).reshape(T * K // CH, CH)
    xp = _make_dispatch()(hidden_states, idx2)
    y = _gemm(te.reshape(G), xp, wgu_bf, wd_bf)
    a, b = _make_combine()(y, dst0.reshape(T // CH, CH),
                           dst1.reshape(T // CH, CH))
    return _final(a, b, w8)


# speedup vs baseline: 1.0500x; 1.0500x over previous
import functools

import jax
import jax.numpy as jnp
from jax import lax
from jax.experimental import pallas as pl
from jax.experimental.pallas import tpu as pltpu
from jax.experimental.pallas import tpu_sc as plsc

T = 4096
D = 1024
F = 512
E = 8
K = 2

BT = 1024
BG = 512
G = T * K // BG + E
P = G * BG

NC = 2
NS = 16
NW = NC * NS
CH = 32


def _router_body(hs_ref, rw_ref, w8_ref, dst0_ref, dst1_ref, te_ref,
                 cnt_ref, e0_ref, e1_ref, r0_ref, r1_ref):
    i = pl.program_id(0)
    nsteps = pl.num_programs(0)

    @pl.when(i == 0)
    def _init():
        cnt_ref[...] = jnp.zeros((1, E), jnp.float32)

    x = hs_ref[...]
    logits = lax.dot_general(x, rw_ref[...], (((1,), (1,)), ((), ())),
                             preferred_element_type=jnp.float32)
    m = jnp.max(logits, axis=-1, keepdims=True)
    ex = jnp.exp(logits - m)
    probs = ex / jnp.sum(ex, axis=-1, keepdims=True)
    cols = lax.broadcasted_iota(jnp.int32, (BT, E), 1)
    i1 = jnp.argmax(probs, axis=-1, keepdims=True)
    is1 = cols == i1
    probs_m = jnp.where(is1, -1.0, probs)
    i2 = jnp.argmax(probs_m, axis=-1, keepdims=True)
    is2 = cols == i2
    w1 = jnp.max(probs, axis=-1, keepdims=True)
    w2 = jnp.max(probs_m, axis=-1, keepdims=True)

    w8_ref[...] = jnp.where(cols == 0, w1, jnp.where(cols == 1, w2, 0.0))

    a1 = is1.astype(jnp.float32)
    a2 = is2.astype(jnp.float32)
    tri = (lax.broadcasted_iota(jnp.int32, (BT, BT), 1)
           < lax.broadcasted_iota(jnp.int32, (BT, BT), 0)).astype(jnp.bfloat16)
    cum1 = jnp.dot(tri, a1.astype(jnp.bfloat16),
                   preferred_element_type=jnp.float32)
    cum2 = jnp.dot(tri, a2.astype(jnp.bfloat16),
                   preferred_element_type=jnp.float32)
    s1 = jnp.sum(a1, axis=0, keepdims=True)
    s2 = jnp.sum(a2, axis=0, keepdims=True)
    cnt = cnt_ref[...]
    rank1 = cnt + cum1
    rank2 = cnt + s1 + cum2
    cnt_ref[...] = cnt + s1 + s2

    e0_ref[i] = a1
    e1_ref[i] = a2
    r0_ref[i] = a1 * rank1
    r1_ref[i] = a2 * rank2

    @pl.when(i == nsteps - 1)
    def _finale():
        counts = cnt_ref[...]
        tiles = jnp.ceil(counts * (1.0 / BG))
        et = lax.broadcasted_iota(jnp.int32, (E, E), 0)
        ee = lax.broadcasted_iota(jnp.int32, (E, E), 1)
        mlt = (et < ee).astype(jnp.float32)
        cum_excl = jnp.dot(tiles, mlt,
                           preferred_element_type=jnp.float32)
        base_rows = cum_excl * float(BG)

        oh0 = e0_ref[...].reshape(T, E)
        oh1 = e1_ref[...].reshape(T, E)
        rf0 = r0_ref[...].reshape(T, E)
        rf1 = r1_ref[...].reshape(T, E)
        dst0_ref[...] = jnp.sum(oh0 * base_rows + rf0, axis=-1,
                                keepdims=True).astype(jnp.int32)
        dst1_ref[...] = jnp.sum(oh1 * base_rows + rf1, axis=-1,
                                keepdims=True).astype(jnp.int32)

        gg = lax.broadcasted_iota(jnp.int32, (G, E), 0).astype(jnp.float32)
        ge = lax.broadcasted_iota(jnp.int32, (G, E), 1).astype(jnp.float32)
        lo = cum_excl
        hi = cum_excl + tiles
        in_e = jnp.logical_and(gg >= lo, gg < hi).astype(jnp.float32)
        te = jnp.sum(in_e * ge, axis=-1, keepdims=True)
        any_e = jnp.sum(in_e, axis=-1, keepdims=True)
        te_ref[...] = jnp.where(any_e > 0.0, te, -1.0).astype(jnp.int32)


def _router(hidden_states, router_weight):
    return pl.pallas_call(
        _router_body,
        grid=(T // BT,),
        in_specs=[
            pl.BlockSpec((BT, D), lambda i: (i, 0)),
            pl.BlockSpec((E, D), lambda i: (0, 0)),
        ],
        out_specs=[
            pl.BlockSpec((BT, E), lambda i: (i, 0)),
            pl.BlockSpec((T, 1), lambda i: (0, 0)),
            pl.BlockSpec((T, 1), lambda i: (0, 0)),
            pl.BlockSpec((G, 1), lambda i: (0, 0)),
        ],
        out_shape=[
            jax.ShapeDtypeStruct((T, E), jnp.float32),
            jax.ShapeDtypeStruct((T, 1), jnp.int32),
            jax.ShapeDtypeStruct((T, 1), jnp.int32),
            jax.ShapeDtypeStruct((G, 1), jnp.int32),
        ],
        scratch_shapes=[
            pltpu.VMEM((1, E), jnp.float32),
            pltpu.VMEM((T // BT, BT, E), jnp.float32),
            pltpu.VMEM((T // BT, BT, E), jnp.float32),
            pltpu.VMEM((T // BT, BT, E), jnp.float32),
            pltpu.VMEM((T // BT, BT, E), jnp.float32),
        ],
        compiler_params=pltpu.CompilerParams(
            dimension_semantics=("arbitrary",)),
    )(hidden_states, router_weight)


@functools.cache
def _make_dispatch():
    mesh = plsc.VectorSubcoreMesh(core_axis_name="c", subcore_axis_name="s")

    nch = T * K // NW // CH

    @functools.partial(
        pl.kernel,
        out_type=jax.ShapeDtypeStruct((P, D), jnp.float32),
        mesh=mesh,
        scratch_types=[
            pltpu.VMEM((nch, CH), jnp.int32),
            pltpu.VMEM((CH, D), jnp.float32),
            pltpu.VMEM((CH, D), jnp.float32),
            pltpu.SemaphoreType.DMA,
            pltpu.SemaphoreType.DMA,
            pltpu.SemaphoreType.DMA,
        ],
    )
    def _dispatch(hid_hbm, idx_hbm, xp_hbm, idx_v, rows0, rows1, ls0, ls1,
                  ss):
        wid = lax.axis_index("s") * NC + lax.axis_index("c")
        npairs = T * K // NW
        tb = (wid % (T // npairs)) * npairs
        rows = (rows0, rows1)
        lsem = (ls0, ls1)
        pltpu.sync_copy(idx_hbm.at[pl.ds(wid * nch, nch)], idx_v)
        cp0 = pltpu.make_async_copy(hid_hbm.at[pl.ds(tb, CH)], rows0, ls0)
        cp0.start()
        cp1 = pltpu.make_async_copy(hid_hbm.at[pl.ds(tb + CH, CH)], rows1,
                                    ls1)
        cp1.start()
        loads = [cp0, cp1]
        for c in range(nch):
            b = c % 2
            loads[b].wait()
            pltpu.async_copy(rows[b], xp_hbm.at[idx_v.at[c]], ss).wait()
            if c + 2 < nch:
                cp = pltpu.make_async_copy(
                    hid_hbm.at[pl.ds(tb + (c + 2) * CH, CH)], rows[b],
                    lsem[b])
                cp.start()
                loads[b] = cp

    return _dispatch


def _gemm_body(te_ref, x_ref, wgu_ref, wd_ref, y_ref, wgu_bf, wd_bf):
    g = pl.program_id(0)
    te_g = te_ref[g]
    changed = jnp.logical_or(g == 0, te_g != te_ref[jnp.maximum(g - 1, 0)])

    @pl.when(jnp.logical_and(te_g >= 0, changed))
    def _cast():
        wgu_bf[...] = wgu_ref[0].astype(jnp.bfloat16)
        wd_bf[...] = wd_ref[0].astype(jnp.bfloat16)

    @pl.when(te_g >= 0)
    def _compute():
        gu = jnp.dot(x_ref[...].astype(jnp.bfloat16), wgu_bf[...],
                     preferred_element_type=jnp.float32)
        gt = gu[:, :F]
        up = gu[:, F:]
        h = (gt * lax.logistic(gt) * up).astype(jnp.bfloat16)
        y_ref[...] = jnp.dot(h, wd_bf[...],
                             preferred_element_type=jnp.float32)


def _gemm(te, xp, wgu_f32, wd_f32):
    def _emap(g, te):
        return (jnp.where(te[g] < 0, E - 1, te[g]), 0, 0)

    grid_spec = pltpu.PrefetchScalarGridSpec(
        num_scalar_prefetch=1,
        grid=(G,),
        in_specs=[
            pl.BlockSpec((BG, D), lambda g, te: (g, 0)),
            pl.BlockSpec((1, D, 2 * F), _emap),
            pl.BlockSpec((1, F, D), _emap),
        ],
        out_specs=pl.BlockSpec((BG, D), lambda g, te: (g, 0)),
        scratch_shapes=[
            pltpu.VMEM((D, 2 * F), jnp.bfloat16),
            pltpu.VMEM((F, D), jnp.bfloat16),
        ],
    )
    return pl.pallas_call(
        _gemm_body,
        grid_spec=grid_spec,
        out_shape=jax.ShapeDtypeStruct((P, D), jnp.float32),
        compiler_params=pltpu.CompilerParams(
            dimension_semantics=("arbitrary",)),
    )(te, xp, wgu_f32, wd_f32)


@functools.cache
def _make_combine():
    mesh = plsc.VectorSubcoreMesh(core_axis_name="c", subcore_axis_name="s")

    nch = T // NW // CH

    @functools.partial(
        pl.kernel,
        out_type=[
            jax.ShapeDtypeStruct((T, D), jnp.float32),
            jax.ShapeDtypeStruct((T, D), jnp.float32),
        ],
        mesh=mesh,
        scratch_types=[
            pltpu.VMEM((nch, CH), jnp.int32),
            pltpu.VMEM((nch, CH), jnp.int32),
            pltpu.VMEM((CH, D), jnp.float32),
            pltpu.VMEM((CH, D), jnp.float32),
            pltpu.SemaphoreType.DMA,
            pltpu.SemaphoreType.DMA,
        ],
    )
    def _combine(y_hbm, d0_hbm, d1_hbm, a_hbm, b_hbm, i0_v, i1_v, rows0,
                 rows1, g0, g1, ):
        wid = lax.axis_index("s") * NC + lax.axis_index("c")
        ntok = T // NW
        rows = (rows0, rows1)
        gsem = (g0, g1)
        pltpu.sync_copy(d0_hbm.at[pl.ds(wid * nch, nch)], i0_v)
        pltpu.sync_copy(d1_hbm.at[pl.ds(wid * nch, nch)], i1_v)
        units = [(kk, cc) for cc in range(nch) for kk in range(2)]

        def _start(u, b):
            kk, cc = units[u]
            iv = i0_v if kk == 0 else i1_v
            cp = pltpu.make_async_copy(y_hbm.at[iv.at[cc]], rows[b],
                                       gsem[b])
            cp.start()
            return cp

        gat = [_start(0, 0), _start(1, 1)]
        for u in range(len(units)):
            b = u % 2
            kk, cc = units[u]
            gat[b].wait()
            dest = a_hbm if kk == 0 else b_hbm
            pltpu.sync_copy(rows[b], dest.at[pl.ds(wid * ntok + cc * CH,
                                                   CH)])
            if u + 2 < len(units):
                gat[b] = _start(u + 2, b)

    return _combine


def _final_body(a_ref, b_ref, w8_ref, out_ref):
    w8 = w8_ref[...]
    cols = lax.broadcasted_iota(jnp.int32, (BT, E), 1)
    w0 = jnp.sum(jnp.where(cols == 0, w8, 0.0), axis=-1, keepdims=True)
    w1 = jnp.sum(jnp.where(cols == 1, w8, 0.0), axis=-1, keepdims=True)
    out_ref[...] = w0 * a_ref[...] + w1 * b_ref[...]


def _final(a, b, w8):
    return pl.pallas_call(
        _final_body,
        grid=(T // BT,),
        in_specs=[
            pl.BlockSpec((BT, D), lambda i: (i, 0)),
            pl.BlockSpec((BT, D), lambda i: (i, 0)),
            pl.BlockSpec((BT, E), lambda i: (i, 0)),
        ],
        out_specs=pl.BlockSpec((BT, D), lambda i: (i, 0)),
        out_shape=jax.ShapeDtypeStruct((T, D), jnp.float32),
    )(a, b, w8)


@jax.jit
def kernel(hidden_states, router_weight, merged_gate_up_proj, merged_down_proj):
    w8, dst0, dst1, te = _router(hidden_states, router_weight)
    idx2 = jnp.concatenate([dst0, dst1], axis=0).reshape(T * K // CH, CH)
    xp = _make_dispatch()(hidden_states, idx2)
    y = _gemm(te.reshape(G), xp, merged_gate_up_proj, merged_down_proj)
    a, b = _make_combine()(y, dst0.reshape(T // CH, CH),
                           dst1.reshape(T // CH, CH))
    return _final(a, b, w8)


# speedup vs baseline: 1.0517x; 1.0016x over previous
import functools

import jax
import jax.numpy as jnp
from jax import lax
from jax.experimental import pallas as pl
from jax.experimental.pallas import tpu as pltpu
from jax.experimental.pallas import tpu_sc as plsc

T = 4096
D = 1024
F = 512
E = 8
K = 2

BT = 1024
BG = 512
G = T * K // BG + E
P = G * BG

NC = 2
NS = 16
NW = NC * NS
CH = 32


def _router_body(hs_ref, rw_ref, w8_ref, dst0_ref, dst1_ref, te_ref,
                 cnt_ref, e0_ref, e1_ref, r0_ref, r1_ref):
    i = pl.program_id(0)
    nsteps = pl.num_programs(0)

    @pl.when(i == 0)
    def _init():
        cnt_ref[...] = jnp.zeros((1, E), jnp.float32)

    x = hs_ref[...]
    logits = lax.dot_general(x, rw_ref[...], (((1,), (1,)), ((), ())),
                             preferred_element_type=jnp.float32)
    m = jnp.max(logits, axis=-1, keepdims=True)
    ex = jnp.exp(logits - m)
    probs = ex / jnp.sum(ex, axis=-1, keepdims=True)
    cols = lax.broadcasted_iota(jnp.int32, (BT, E), 1)
    i1 = jnp.argmax(probs, axis=-1, keepdims=True)
    is1 = cols == i1
    probs_m = jnp.where(is1, -1.0, probs)
    i2 = jnp.argmax(probs_m, axis=-1, keepdims=True)
    is2 = cols == i2
    w1 = jnp.max(probs, axis=-1, keepdims=True)
    w2 = jnp.max(probs_m, axis=-1, keepdims=True)

    w8_ref[...] = jnp.where(cols == 0, w1, jnp.where(cols == 1, w2, 0.0))

    a1 = is1.astype(jnp.float32)
    a2 = is2.astype(jnp.float32)
    tri = (lax.broadcasted_iota(jnp.int32, (BT, BT), 1)
           < lax.broadcasted_iota(jnp.int32, (BT, BT), 0)).astype(jnp.bfloat16)
    cum1 = jnp.dot(tri, a1.astype(jnp.bfloat16),
                   preferred_element_type=jnp.float32)
    cum2 = jnp.dot(tri, a2.astype(jnp.bfloat16),
                   preferred_element_type=jnp.float32)
    s1 = jnp.sum(a1, axis=0, keepdims=True)
    s2 = jnp.sum(a2, axis=0, keepdims=True)
    cnt = cnt_ref[...]
    rank1 = cnt + cum1
    rank2 = cnt + s1 + cum2
    cnt_ref[...] = cnt + s1 + s2

    e0_ref[i] = a1
    e1_ref[i] = a2
    r0_ref[i] = a1 * rank1
    r1_ref[i] = a2 * rank2

    @pl.when(i == nsteps - 1)
    def _finale():
        counts = cnt_ref[...]
        tiles = jnp.ceil(counts * (1.0 / BG))
        et = lax.broadcasted_iota(jnp.int32, (E, E), 0)
        ee = lax.broadcasted_iota(jnp.int32, (E, E), 1)
        mlt = (et < ee).astype(jnp.float32)
        cum_excl = jnp.dot(tiles, mlt,
                           preferred_element_type=jnp.float32)
        base_rows = cum_excl * float(BG)

        oh0 = e0_ref[...].reshape(T, E)
        oh1 = e1_ref[...].reshape(T, E)
        rf0 = r0_ref[...].reshape(T, E)
        rf1 = r1_ref[...].reshape(T, E)
        dst0_ref[...] = jnp.sum(oh0 * base_rows + rf0, axis=-1,
                                keepdims=True).astype(jnp.int32)
        dst1_ref[...] = jnp.sum(oh1 * base_rows + rf1, axis=-1,
                                keepdims=True).astype(jnp.int32)

        gg = lax.broadcasted_iota(jnp.int32, (G, E), 0).astype(jnp.float32)
        ge = lax.broadcasted_iota(jnp.int32, (G, E), 1).astype(jnp.float32)
        lo = cum_excl
        hi = cum_excl + tiles
        in_e = jnp.logical_and(gg >= lo, gg < hi).astype(jnp.float32)
        te = jnp.sum(in_e * ge, axis=-1, keepdims=True)
        any_e = jnp.sum(in_e, axis=-1, keepdims=True)
        te_ref[...] = jnp.where(any_e > 0.0, te, -1.0).astype(jnp.int32)


def _router(hidden_states, router_weight):
    return pl.pallas_call(
        _router_body,
        grid=(T // BT,),
        in_specs=[
            pl.BlockSpec((BT, D), lambda i: (i, 0)),
            pl.BlockSpec((E, D), lambda i: (0, 0)),
        ],
        out_specs=[
            pl.BlockSpec((BT, E), lambda i: (i, 0)),
            pl.BlockSpec((T, 1), lambda i: (0, 0)),
            pl.BlockSpec((T, 1), lambda i: (0, 0)),
            pl.BlockSpec((G, 1), lambda i: (0, 0)),
        ],
        out_shape=[
            jax.ShapeDtypeStruct((T, E), jnp.float32),
            jax.ShapeDtypeStruct((T, 1), jnp.int32),
            jax.ShapeDtypeStruct((T, 1), jnp.int32),
            jax.ShapeDtypeStruct((G, 1), jnp.int32),
        ],
        scratch_shapes=[
            pltpu.VMEM((1, E), jnp.float32),
            pltpu.VMEM((T // BT, BT, E), jnp.float32),
            pltpu.VMEM((T // BT, BT, E), jnp.float32),
            pltpu.VMEM((T // BT, BT, E), jnp.float32),
            pltpu.VMEM((T // BT, BT, E), jnp.float32),
        ],
        compiler_params=pltpu.CompilerParams(
            dimension_semantics=("arbitrary",)),
    )(hidden_states, router_weight)


@functools.cache
def _make_dispatch():
    mesh = plsc.VectorSubcoreMesh(core_axis_name="c", subcore_axis_name="s")

    nch = T * K // NW // CH

    @functools.partial(
        pl.kernel,
        out_type=jax.ShapeDtypeStruct((P, D), jnp.float32),
        mesh=mesh,
        scratch_types=[
            pltpu.VMEM((nch, CH), jnp.int32),
            pltpu.VMEM((CH, D), jnp.float32),
            pltpu.VMEM((CH, D), jnp.float32),
            pltpu.SemaphoreType.DMA,
            pltpu.SemaphoreType.DMA,
            pltpu.SemaphoreType.DMA,
        ],
    )
    def _dispatch(hid_hbm, i0_hbm, i1_hbm, xp_hbm, idx_v, rows0, rows1,
                  ls0, ls1, ss):
        wid = lax.axis_index("s") * NC + lax.axis_index("c")
        npairs = T * K // NW
        tb = (wid % (T // npairs)) * npairs
        rows = (rows0, rows1)
        lsem = (ls0, ls1)
        half = T // npairs
        irow = (wid % half) * nch

        @pl.when(wid < half)
        def _load0():
            pltpu.sync_copy(i0_hbm.at[pl.ds(irow, nch)], idx_v)

        @pl.when(wid >= half)
        def _load1():
            pltpu.sync_copy(i1_hbm.at[pl.ds(irow, nch)], idx_v)
        cp0 = pltpu.make_async_copy(hid_hbm.at[pl.ds(tb, CH)], rows0, ls0)
        cp0.start()
        cp1 = pltpu.make_async_copy(hid_hbm.at[pl.ds(tb + CH, CH)], rows1,
                                    ls1)
        cp1.start()
        loads = [cp0, cp1]
        for c in range(nch):
            b = c % 2
            loads[b].wait()
            pltpu.async_copy(rows[b], xp_hbm.at[idx_v.at[c]], ss).wait()
            if c + 2 < nch:
                cp = pltpu.make_async_copy(
                    hid_hbm.at[pl.ds(tb + (c + 2) * CH, CH)], rows[b],
                    lsem[b])
                cp.start()
                loads[b] = cp

    return _dispatch


def _gemm_body(te_ref, x_ref, wgu_ref, wd_ref, y_ref, wgu_bf, wd_bf):
    g = pl.program_id(0)
    te_g = te_ref[g]
    changed = jnp.logical_or(g == 0, te_g != te_ref[jnp.maximum(g - 1, 0)])

    @pl.when(jnp.logical_and(te_g >= 0, changed))
    def _cast():
        wgu_bf[...] = wgu_ref[0].astype(jnp.bfloat16)
        wd_bf[...] = wd_ref[0].astype(jnp.bfloat16)

    @pl.when(te_g >= 0)
    def _compute():
        gu = jnp.dot(x_ref[...].astype(jnp.bfloat16), wgu_bf[...],
                     preferred_element_type=jnp.float32)
        gt = gu[:, :F]
        up = gu[:, F:]
        h = (gt * lax.logistic(gt) * up).astype(jnp.bfloat16)
        y_ref[...] = jnp.dot(h, wd_bf[...],
                             preferred_element_type=jnp.float32)


def _gemm(te, xp, wgu_f32, wd_f32):
    def _emap(g, te):
        return (jnp.where(te[g] < 0, E - 1, te[g]), 0, 0)

    grid_spec = pltpu.PrefetchScalarGridSpec(
        num_scalar_prefetch=1,
        grid=(G,),
        in_specs=[
            pl.BlockSpec((BG, D), lambda g, te: (g, 0)),
            pl.BlockSpec((1, D, 2 * F), _emap),
            pl.BlockSpec((1, F, D), _emap),
        ],
        out_specs=pl.BlockSpec((BG, D), lambda g, te: (g, 0)),
        scratch_shapes=[
            pltpu.VMEM((D, 2 * F), jnp.bfloat16),
            pltpu.VMEM((F, D), jnp.bfloat16),
        ],
    )
    return pl.pallas_call(
        _gemm_body,
        grid_spec=grid_spec,
        out_shape=jax.ShapeDtypeStruct((P, D), jnp.float32),
        compiler_params=pltpu.CompilerParams(
            dimension_semantics=("arbitrary",)),
    )(te, xp, wgu_f32, wd_f32)


@functools.cache
def _make_combine():
    mesh = plsc.VectorSubcoreMesh(core_axis_name="c", subcore_axis_name="s")

    nch = T // NW // CH

    @functools.partial(
        pl.kernel,
        out_type=[
            jax.ShapeDtypeStruct((T, D), jnp.float32),
            jax.ShapeDtypeStruct((T, D), jnp.float32),
        ],
        mesh=mesh,
        scratch_types=[
            pltpu.VMEM((nch, CH), jnp.int32),
            pltpu.VMEM((nch, CH), jnp.int32),
            pltpu.VMEM((CH, D), jnp.float32),
            pltpu.VMEM((CH, D), jnp.float32),
            pltpu.SemaphoreType.DMA,
            pltpu.SemaphoreType.DMA,
        ],
    )
    def _combine(y_hbm, d0_hbm, d1_hbm, a_hbm, b_hbm, i0_v, i1_v, rows0,
                 rows1, g0, g1, ):
        wid = lax.axis_index("s") * NC + lax.axis_index("c")
        ntok = T // NW
        rows = (rows0, rows1)
        gsem = (g0, g1)
        pltpu.sync_copy(d0_hbm.at[pl.ds(wid * nch, nch)], i0_v)
        pltpu.sync_copy(d1_hbm.at[pl.ds(wid * nch, nch)], i1_v)
        units = [(kk, cc) for cc in range(nch) for kk in range(2)]

        def _start(u, b):
            kk, cc = units[u]
            iv = i0_v if kk == 0 else i1_v
            cp = pltpu.make_async_copy(y_hbm.at[iv.at[cc]], rows[b],
                                       gsem[b])
            cp.start()
            return cp

        gat = [_start(0, 0), _start(1, 1)]
        for u in range(len(units)):
            b = u % 2
            kk, cc = units[u]
            gat[b].wait()
            dest = a_hbm if kk == 0 else b_hbm
            pltpu.sync_copy(rows[b], dest.at[pl.ds(wid * ntok + cc * CH,
                                                   CH)])
            if u + 2 < len(units):
                gat[b] = _start(u + 2, b)

    return _combine


def _final_body(a_ref, b_ref, w8_ref, out_ref):
    w8 = w8_ref[...]
    cols = lax.broadcasted_iota(jnp.int32, (BT, E), 1)
    w0 = jnp.sum(jnp.where(cols == 0, w8, 0.0), axis=-1, keepdims=True)
    w1 = jnp.sum(jnp.where(cols == 1, w8, 0.0), axis=-1, keepdims=True)
    out_ref[...] = w0 * a_ref[...] + w1 * b_ref[...]


def _final(a, b, w8):
    return pl.pallas_call(
        _final_body,
        grid=(T // BT,),
        in_specs=[
            pl.BlockSpec((BT, D), lambda i: (i, 0)),
            pl.BlockSpec((BT, D), lambda i: (i, 0)),
            pl.BlockSpec((BT, E), lambda i: (i, 0)),
        ],
        out_specs=pl.BlockSpec((BT, D), lambda i: (i, 0)),
        out_shape=jax.ShapeDtypeStruct((T, D), jnp.float32),
    )(a, b, w8)


@jax.jit
def kernel(hidden_states, router_weight, merged_gate_up_proj, merged_down_proj):
    w8, dst0, dst1, te = _router(hidden_states, router_weight)
    d0 = dst0.reshape(T // CH, CH)
    d1 = dst1.reshape(T // CH, CH)
    xp = _make_dispatch()(hidden_states, d0, d1)
    y = _gemm(te.reshape(G), xp, merged_gate_up_proj, merged_down_proj)
    a, b = _make_combine()(y, d0, d1)
    return _final(a, b, w8)
